# Initial kernel scaffold; baseline (speedup 1.0000x reference)
import functools

import jax
import jax.numpy as jnp
from jax import lax
from jax.experimental import pallas as pl
from jax.experimental.pallas import tpu as pltpu
from jax.experimental.pallas import tpu_sc as plsc

N = 10000
D = 128
E = 320000
R = E // 128
R_SC = R // 2
RPT = R_SC // 16
REM = R_SC - RPT * 16
NPT = N // 16

_MESH = plsc.VectorSubcoreMesh(core_axis_name="c", subcore_axis_name="s")


def _tile_rows(s):
    row0 = RPT * s + jnp.maximum(s - (16 - REM), 0)
    nrows = RPT + (s >= (16 - REM)).astype(jnp.int32)
    return row0, nrows


@functools.partial(
    pl.kernel,
    mesh=_MESH,
    out_type=jax.ShapeDtypeStruct((2, N, 16), jnp.float32),
    scratch_types=[
        pltpu.VMEM((79, 128), jnp.int32),
        pltpu.VMEM((128, 16), jnp.float32),
        pltpu.VMEM((125, 16), jnp.float32),
        pltpu.VMEM_SHARED((N, 16), jnp.float32),
    ],
)
def _deg_hist(dst_hbm, out_hbm, dbuf, ones, zbuf, acc):
    c = lax.axis_index("c")
    s = lax.axis_index("s")
    hot = jnp.where(lax.iota(jnp.int32, 16) == 0, 1.0, 0.0).astype(jnp.float32)
    zero = jnp.zeros((16,), jnp.float32)

    def init(i, _):
        ones[i, :] = hot
        return 0
    lax.fori_loop(0, 128, init, 0)

    def zinit(i, _):
        zbuf[i, :] = zero
        return 0
    lax.fori_loop(0, 125, zinit, 0)

    r0 = s * NPT
    for q in range(5):
        pltpu.sync_copy(zbuf, acc.at[pl.ds(r0 + q * 125, 125), :])

    row0_l, nrows = _tile_rows(s)
    row0 = c * R_SC + row0_l
    pltpu.sync_copy(dst_hbm.at[pl.ds(row0, 79), :], dbuf)
    plsc.subcore_barrier()

    def body(i, _):
        pltpu.sync_copy(ones, acc.at[dbuf.at[i]], add=True)
        return 0
    lax.fori_loop(0, nrows, body, 0)
    plsc.subcore_barrier()

    pltpu.sync_copy(acc.at[pl.ds(r0, NPT), :], out_hbm.at[c, pl.ds(r0, NPT), :])


@functools.partial(
    pl.kernel,
    mesh=_MESH,
    out_type=jax.ShapeDtypeStruct((2, N, D), jnp.float32),
    scratch_types=[
        pltpu.VMEM((79, 128), jnp.int32),
        pltpu.VMEM((79, 128), jnp.int32),
        pltpu.VMEM((128, D), jnp.float32),
        pltpu.VMEM((125, D), jnp.float32),
        pltpu.VMEM_SHARED((N, D), jnp.float32),
        pltpu.SemaphoreType.DMA,
    ],
)
def _edge_pass(h_hbm, src_hbm, dst_hbm, out_hbm, sbuf, dbuf, rows, zbuf, acc, sem):
    c = lax.axis_index("c")
    s = lax.axis_index("s")
    zero = jnp.zeros((16,), jnp.float32)

    def zinit(i, _):
        for b in range(D // 16):
            zbuf[i, pl.ds(b * 16, 16)] = zero
        return 0
    lax.fori_loop(0, 125, zinit, 0)

    r0 = s * NPT
    for q in range(5):
        pltpu.sync_copy(zbuf, acc.at[pl.ds(r0 + q * 125, 125), :])

    row0_l, nrows = _tile_rows(s)
    row0 = c * R_SC + row0_l
    pltpu.sync_copy(src_hbm.at[pl.ds(row0, 79), :], sbuf)
    pltpu.sync_copy(dst_hbm.at[pl.ds(row0, 79), :], dbuf)
    plsc.subcore_barrier()

    def body(i, _):
        pltpu.async_copy(h_hbm.at[sbuf.at[i]], rows, sem).wait()
        pltpu.sync_copy(rows, acc.at[dbuf.at[i]], add=True)
        return 0
    lax.fori_loop(0, nrows, body, 0)
    plsc.subcore_barrier()

    pltpu.sync_copy(acc.at[pl.ds(r0, NPT), :], out_hbm.at[c, pl.ds(r0, NPT), :])


_BLK = 1000
_GRID = N // _BLK


def _dinv_of(deg2_blk):
    deg = jnp.sum(deg2_blk[0], axis=-1) + jnp.sum(deg2_blk[1], axis=-1)
    return jnp.where(deg > 0, 1.0 / jnp.sqrt(jnp.maximum(deg, 1.0)), 0.0)


def _tc1_body(deg2_ref, x_ref, w_ref, o_ref):
    dinv = _dinv_of(deg2_ref[...])
    h = jnp.dot(x_ref[...], w_ref[...], preferred_element_type=jnp.float32)
    o_ref[...] = h * dinv[:, None]


def _tc2_body(s2_ref, deg2_ref, b_ref, w_ref, o_ref):
    dinv = _dinv_of(deg2_ref[...])
    t = s2_ref[0] + s2_ref[1]
    t = jnp.maximum(t * dinv[:, None] + b_ref[...], 0.0)
    h = jnp.dot(t, w_ref[...], preferred_element_type=jnp.float32)
    o_ref[...] = h * dinv[:, None]


def _tc3_body(s2_ref, deg2_ref, b_ref, o_ref):
    dinv = _dinv_of(deg2_ref[...])
    t = s2_ref[0] + s2_ref[1]
    o_ref[...] = jnp.maximum(t * dinv[:, None] + b_ref[...], 0.0)


_deg_spec = pl.BlockSpec((2, _BLK, 16), lambda i: (0, i, 0))
_row_spec = pl.BlockSpec((_BLK, D), lambda i: (i, 0))
_s_spec = pl.BlockSpec((2, _BLK, D), lambda i: (0, i, 0))
_w_spec = pl.BlockSpec((D, D), lambda i: (0, 0))
_b_spec = pl.BlockSpec((1, D), lambda i: (0, 0))
_out_t = jax.ShapeDtypeStruct((N, D), jnp.float32)

_tc1 = pl.pallas_call(
    _tc1_body, grid=(_GRID,),
    in_specs=[_deg_spec, _row_spec, _w_spec],
    out_specs=_row_spec, out_shape=_out_t)

_tc2 = pl.pallas_call(
    _tc2_body, grid=(_GRID,),
    in_specs=[_s_spec, _deg_spec, _b_spec, _w_spec],
    out_specs=_row_spec, out_shape=_out_t)

_tc3 = pl.pallas_call(
    _tc3_body, grid=(_GRID,),
    in_specs=[_s_spec, _deg_spec, _b_spec],
    out_specs=_row_spec, out_shape=_out_t)


def kernel(x, edge_index, W1, b1, W2, b2):
    src2d = edge_index[0].reshape(R, 128)
    dst2d = edge_index[1].reshape(R, 128)
    deg2 = _deg_hist(dst2d)
    h1 = _tc1(deg2, x, W1)
    s1 = _edge_pass(h1, src2d, dst2d)
    h2 = _tc2(s1, deg2, b1.reshape(1, D), W2)
    s2 = _edge_pass(h2, src2d, dst2d)
    return _tc3(s2, deg2, b2.reshape(1, D))


# speedup vs baseline: 16.0470x; 16.0470x over previous
import functools

import jax
import jax.numpy as jnp
import numpy as np
from jax import lax
from jax.experimental import pallas as pl
from jax.experimental.pallas import tpu as pltpu
from jax.experimental.pallas import tpu_sc as plsc

N = 10000
D = 128
E = 320000
R = E // 128
RPT = 80
PAD_E = 32 * RPT * 128 - E
N_PAD = 10240
NPT = N_PAD // 16

_MESH = plsc.VectorSubcoreMesh(core_axis_name="c", subcore_axis_name="s")

_PAD_SRC = np.arange(PAD_E, dtype=np.int32) % N
_PAD_DST = N + (np.arange(PAD_E, dtype=np.int32) % (N_PAD - N))

_ONESD = np.ones((128, D), np.float32)
_ZEROD = np.zeros((128, D), np.float32)


@functools.partial(
    pl.kernel,
    mesh=_MESH,
    out_type=jax.ShapeDtypeStruct((2, N_PAD, D), jnp.float32),
    scratch_types=[
        pltpu.VMEM((RPT, 128), jnp.int32),
        pltpu.VMEM((128, D), jnp.float32),
        pltpu.VMEM_SHARED((N_PAD, D), jnp.float32),
    ],
)
def _deg_hist(dst_hbm, ones_hbm, zero_hbm, out_hbm, dbuf, ones, acc):
    c = lax.axis_index("c")
    s = lax.axis_index("s")
    w = c * 16 + s

    pltpu.sync_copy(ones_hbm, ones)
    r0 = s * NPT
    for q in range(5):
        pltpu.sync_copy(zero_hbm, acc.at[pl.ds(r0 + q * 128, 128), :])

    pltpu.sync_copy(dst_hbm.at[w], dbuf)
    plsc.subcore_barrier()

    def body(i, _):
        pltpu.sync_copy(ones, acc.at[dbuf.at[i]], add=True)
        return 0
    lax.fori_loop(0, RPT, body, 0)
    plsc.subcore_barrier()

    pltpu.sync_copy(acc.at[pl.ds(r0, NPT), :], out_hbm.at[c, pl.ds(r0, NPT), :])


@functools.partial(
    pl.kernel,
    mesh=_MESH,
    out_type=jax.ShapeDtypeStruct((2, N_PAD, D), jnp.float32),
    scratch_types=[
        pltpu.VMEM((RPT, 128), jnp.int32),
        pltpu.VMEM((RPT, 128), jnp.int32),
        pltpu.VMEM((128, D), jnp.float32),
        pltpu.VMEM_SHARED((N_PAD, D), jnp.float32),
        pltpu.SemaphoreType.DMA,
    ],
)
def _edge_pass(h_hbm, src_hbm, dst_hbm, zero_hbm, out_hbm, sbuf, dbuf, rows, acc, sem):
    c = lax.axis_index("c")
    s = lax.axis_index("s")
    w = c * 16 + s

    r0 = s * NPT
    for q in range(5):
        pltpu.sync_copy(zero_hbm, acc.at[pl.ds(r0 + q * 128, 128), :])

    pltpu.sync_copy(src_hbm.at[w], sbuf)
    pltpu.sync_copy(dst_hbm.at[w], dbuf)
    plsc.subcore_barrier()

    def body(i, _):
        pltpu.async_copy(h_hbm.at[sbuf.at[i]], rows, sem).wait()
        pltpu.sync_copy(rows, acc.at[dbuf.at[i]], add=True)
        return 0
    lax.fori_loop(0, RPT, body, 0)
    plsc.subcore_barrier()

    pltpu.sync_copy(acc.at[pl.ds(r0, NPT), :], out_hbm.at[c, pl.ds(r0, NPT), :])


_BLK = 1000
_GRID = N // _BLK


def _dinv_of(deg2_blk):
    deg = deg2_blk[0, :, 0] + deg2_blk[1, :, 0]
    return jnp.where(deg > 0, 1.0 / jnp.sqrt(jnp.maximum(deg, 1.0)), 0.0)


def _tc1_body(deg2_ref, x_ref, w_ref, o_ref):
    dinv = _dinv_of(deg2_ref[...])
    h = jnp.dot(x_ref[...], w_ref[...], preferred_element_type=jnp.float32)
    o_ref[...] = h * dinv[:, None]


def _tc2_body(s2_ref, deg2_ref, b_ref, w_ref, o_ref):
    dinv = _dinv_of(deg2_ref[...])
    t = s2_ref[0] + s2_ref[1]
    t = jnp.maximum(t * dinv[:, None] + b_ref[...], 0.0)
    h = jnp.dot(t, w_ref[...], preferred_element_type=jnp.float32)
    o_ref[...] = h * dinv[:, None]


def _tc3_body(s2_ref, deg2_ref, b_ref, o_ref):
    dinv = _dinv_of(deg2_ref[...])
    t = s2_ref[0] + s2_ref[1]
    o_ref[...] = jnp.maximum(t * dinv[:, None] + b_ref[...], 0.0)


_deg_spec = pl.BlockSpec((2, _BLK, D), lambda i: (0, i, 0))
_row_spec = pl.BlockSpec((_BLK, D), lambda i: (i, 0))
_s_spec = pl.BlockSpec((2, _BLK, D), lambda i: (0, i, 0))
_w_spec = pl.BlockSpec((D, D), lambda i: (0, 0))
_b_spec = pl.BlockSpec((1, D), lambda i: (0, 0))
_out_t = jax.ShapeDtypeStruct((N, D), jnp.float32)

_tc1 = pl.pallas_call(
    _tc1_body, grid=(_GRID,),
    in_specs=[_deg_spec, _row_spec, _w_spec],
    out_specs=_row_spec, out_shape=_out_t)

_tc2 = pl.pallas_call(
    _tc2_body, grid=(_GRID,),
    in_specs=[_s_spec, _deg_spec, _b_spec, _w_spec],
    out_specs=_row_spec, out_shape=_out_t)

_tc3 = pl.pallas_call(
    _tc3_body, grid=(_GRID,),
    in_specs=[_s_spec, _deg_spec, _b_spec],
    out_specs=_row_spec, out_shape=_out_t)


def kernel(x, edge_index, W1, b1, W2, b2):
    src3d = jnp.concatenate(
        [edge_index[0], jnp.asarray(_PAD_SRC)]).reshape(32, RPT, 128)
    dst3d = jnp.concatenate(
        [edge_index[1], jnp.asarray(_PAD_DST)]).reshape(32, RPT, 128)
    ones_c = jnp.asarray(_ONESD)
    zero_c = jnp.asarray(_ZEROD)
    deg2 = _deg_hist(dst3d, ones_c, zero_c)
    h1 = _tc1(deg2, x, W1)
    s1 = _edge_pass(h1, src3d, dst3d, zero_c)
    h2 = _tc2(s1, deg2, b1.reshape(1, D), W2)
    s2 = _edge_pass(h2, src3d, dst3d, zero_c)
    return _tc3(s2, deg2, b2.reshape(1, D))


# speedup vs baseline: 19.3066x; 1.2031x over previous
import functools

import jax
import jax.numpy as jnp
import numpy as np
from jax import lax
from jax.experimental import pallas as pl
from jax.experimental.pallas import tpu as pltpu
from jax.experimental.pallas import tpu_sc as plsc

N = 10000
D = 128
E = 320000
R = E // 128
RPT = 80
PAD_E = 32 * RPT * 128 - E
N_PAD = 10240
NPT = N_PAD // 16

_MESH = plsc.VectorSubcoreMesh(core_axis_name="c", subcore_axis_name="s")

_PAD_SRC = np.arange(PAD_E, dtype=np.int32) % N
_PAD_DST = N + (np.arange(PAD_E, dtype=np.int32) % (N_PAD - N))

_ONESD = np.ones((128, D), np.float32)
_ZEROD = np.zeros((128, D), np.float32)


@functools.partial(
    pl.kernel,
    mesh=_MESH,
    out_type=jax.ShapeDtypeStruct((2, N_PAD, D), jnp.float32),
    scratch_types=[
        pltpu.VMEM((RPT, 128), jnp.int32),
        pltpu.VMEM((128, D), jnp.float32),
        pltpu.VMEM_SHARED((N_PAD, D), jnp.float32),
        pltpu.SemaphoreType.DMA,
    ],
)
def _deg_hist(dst_hbm, ones_hbm, zero_hbm, out_hbm, dbuf, ones, acc, ssem):
    c = lax.axis_index("c")
    s = lax.axis_index("s")
    w = c * 16 + s

    pltpu.sync_copy(ones_hbm, ones)
    r0 = s * NPT
    for q in range(5):
        pltpu.sync_copy(zero_hbm, acc.at[pl.ds(r0 + q * 128, 128), :])

    pltpu.sync_copy(dst_hbm.at[w], dbuf)
    plsc.subcore_barrier()

    def body(i, _):
        @pl.when(i >= 4)
        def _():
            pltpu.make_async_copy(ones, acc.at[dbuf.at[0]], ssem).wait()
        pltpu.async_copy(ones, acc.at[dbuf.at[i]], ssem, add=True)
        return 0
    lax.fori_loop(0, RPT, body, 0)
    for _ in range(4):
        pltpu.make_async_copy(ones, acc.at[dbuf.at[0]], ssem).wait()
    plsc.subcore_barrier()

    pltpu.sync_copy(acc.at[pl.ds(r0, NPT), :], out_hbm.at[c, pl.ds(r0, NPT), :])


@functools.partial(
    pl.kernel,
    mesh=_MESH,
    out_type=jax.ShapeDtypeStruct((2, N_PAD, D), jnp.float32),
    scratch_types=[
        pltpu.VMEM((40, 128), jnp.int32),
        pltpu.VMEM((40, 128), jnp.int32),
        pltpu.VMEM((128, D), jnp.float32),
        pltpu.VMEM((128, D), jnp.float32),
        pltpu.VMEM_SHARED((N_PAD, D), jnp.float32),
        pltpu.SemaphoreType.DMA,
        pltpu.SemaphoreType.DMA,
        pltpu.SemaphoreType.DMA,
        pltpu.SemaphoreType.DMA,
    ],
)
def _edge_pass(h_hbm, src_hbm, dst_hbm, zero_hbm, out_hbm,
               sbuf, dbuf, rows0, rows1, acc, gs0, gs1, ss0, ss1):
    c = lax.axis_index("c")
    s = lax.axis_index("s")
    w = c * 16 + s
    C = 40

    r0 = s * NPT
    for q in range(5):
        pltpu.sync_copy(zero_hbm, acc.at[pl.ds(r0 + q * 128, 128), :])
    plsc.subcore_barrier()

    for p in range(RPT // C):
        pltpu.sync_copy(src_hbm.at[w, pl.ds(p * C, C), :], sbuf)
        pltpu.sync_copy(dst_hbm.at[w, pl.ds(p * C, C), :], dbuf)
        pltpu.async_copy(h_hbm.at[sbuf.at[0]], rows0, gs0)

        def pair(j, _):
            i0 = 2 * j
            i1 = 2 * j + 1
            pltpu.make_async_copy(h_hbm.at[sbuf.at[0]], rows0, gs0).wait()
            pltpu.async_copy(rows0, acc.at[dbuf.at[i0]], ss0, add=True)

            @pl.when(j > 0)
            def _():
                pltpu.make_async_copy(rows1, acc.at[dbuf.at[0]], ss1).wait()
            pltpu.async_copy(h_hbm.at[sbuf.at[i1]], rows1, gs1)
            pltpu.make_async_copy(h_hbm.at[sbuf.at[0]], rows1, gs1).wait()
            pltpu.async_copy(rows1, acc.at[dbuf.at[i1]], ss1, add=True)
            pltpu.make_async_copy(rows0, acc.at[dbuf.at[0]], ss0).wait()
            nxt = jnp.minimum(i0 + 2, C - 1)
            pltpu.async_copy(h_hbm.at[sbuf.at[nxt]], rows0, gs0)
            return 0
        lax.fori_loop(0, C // 2, pair, 0)
        pltpu.make_async_copy(rows1, acc.at[dbuf.at[0]], ss1).wait()
        pltpu.make_async_copy(h_hbm.at[sbuf.at[0]], rows0, gs0).wait()
    plsc.subcore_barrier()

    pltpu.sync_copy(acc.at[pl.ds(r0, NPT), :], out_hbm.at[c, pl.ds(r0, NPT), :])


_BLK = 1000
_GRID = N // _BLK


def _dinv_of(deg2_blk):
    deg = deg2_blk[0, :, 0] + deg2_blk[1, :, 0]
    return jnp.where(deg > 0, 1.0 / jnp.sqrt(jnp.maximum(deg, 1.0)), 0.0)


def _tc1_body(deg2_ref, x_ref, w_ref, o_ref):
    dinv = _dinv_of(deg2_ref[...])
    h = jnp.dot(x_ref[...], w_ref[...], preferred_element_type=jnp.float32)
    o_ref[...] = h * dinv[:, None]


def _tc2_body(s2_ref, deg2_ref, b_ref, w_ref, o_ref):
    dinv = _dinv_of(deg2_ref[...])
    t = s2_ref[0] + s2_ref[1]
    t = jnp.maximum(t * dinv[:, None] + b_ref[...], 0.0)
    h = jnp.dot(t, w_ref[...], preferred_element_type=jnp.float32)
    o_ref[...] = h * dinv[:, None]


def _tc3_body(s2_ref, deg2_ref, b_ref, o_ref):
    dinv = _dinv_of(deg2_ref[...])
    t = s2_ref[0] + s2_ref[1]
    o_ref[...] = jnp.maximum(t * dinv[:, None] + b_ref[...], 0.0)


_deg_spec = pl.BlockSpec((2, _BLK, D), lambda i: (0, i, 0))
_row_spec = pl.BlockSpec((_BLK, D), lambda i: (i, 0))
_s_spec = pl.BlockSpec((2, _BLK, D), lambda i: (0, i, 0))
_w_spec = pl.BlockSpec((D, D), lambda i: (0, 0))
_b_spec = pl.BlockSpec((1, D), lambda i: (0, 0))
_out_t = jax.ShapeDtypeStruct((N, D), jnp.float32)

_tc1 = pl.pallas_call(
    _tc1_body, grid=(_GRID,),
    in_specs=[_deg_spec, _row_spec, _w_spec],
    out_specs=_row_spec, out_shape=_out_t)

_tc2 = pl.pallas_call(
    _tc2_body, grid=(_GRID,),
    in_specs=[_s_spec, _deg_spec, _b_spec, _w_spec],
    out_specs=_row_spec, out_shape=_out_t)

_tc3 = pl.pallas_call(
    _tc3_body, grid=(_GRID,),
    in_specs=[_s_spec, _deg_spec, _b_spec],
    out_specs=_row_spec, out_shape=_out_t)


def kernel(x, edge_index, W1, b1, W2, b2):
    src3d = jnp.concatenate(
        [edge_index[0], jnp.asarray(_PAD_SRC)]).reshape(32, RPT, 128)
    dst3d = jnp.concatenate(
        [edge_index[1], jnp.asarray(_PAD_DST)]).reshape(32, RPT, 128)
    ones_c = jnp.asarray(_ONESD)
    zero_c = jnp.asarray(_ZEROD)
    deg2 = _deg_hist(dst3d, ones_c, zero_c)
    h1 = _tc1(deg2, x, W1)
    s1 = _edge_pass(h1, src3d, dst3d, zero_c)
    h2 = _tc2(s1, deg2, b1.reshape(1, D), W2)
    s2 = _edge_pass(h2, src3d, dst3d, zero_c)
    return _tc3(s2, deg2, b2.reshape(1, D))


# speedup vs baseline: 19.7770x; 1.0244x over previous
import functools

import jax
import jax.numpy as jnp
import numpy as np
from jax import lax
from jax.experimental import pallas as pl
from jax.experimental.pallas import tpu as pltpu
from jax.experimental.pallas import tpu_sc as plsc

N = 10000
D = 128
E = 320000
R = E // 128
RPT = 80
PAD_E = 32 * RPT * 128 - E
N_PAD = 10240
NPT = N_PAD // 16

_MESH = plsc.VectorSubcoreMesh(core_axis_name="c", subcore_axis_name="s")

_PAD_SRC = np.arange(PAD_E, dtype=np.int32) % N
_PAD_DST = N + (np.arange(PAD_E, dtype=np.int32) % (N_PAD - N))

_ONESD = np.ones((128, D), np.float32)
_ZEROD = np.zeros((128, D), np.float32)


@functools.partial(
    pl.kernel,
    mesh=_MESH,
    out_type=jax.ShapeDtypeStruct((2, N_PAD, D), jnp.float32),
    scratch_types=[
        pltpu.VMEM((RPT, 128), jnp.int32),
        pltpu.VMEM((128, D), jnp.float32),
        pltpu.VMEM_SHARED((N_PAD, D), jnp.float32),
        pltpu.SemaphoreType.DMA,
    ],
)
def _deg_hist(dst_hbm, ones_hbm, zero_hbm, out_hbm, dbuf, ones, acc, ssem):
    c = lax.axis_index("c")
    s = lax.axis_index("s")
    w = c * 16 + s

    pltpu.sync_copy(ones_hbm, ones)
    r0 = s * NPT
    for q in range(5):
        pltpu.sync_copy(zero_hbm, acc.at[pl.ds(r0 + q * 128, 128), :])

    pltpu.sync_copy(dst_hbm.at[w], dbuf)
    plsc.subcore_barrier()

    def body(i, _):
        @pl.when(i >= 4)
        def _():
            pltpu.make_async_copy(ones, acc.at[dbuf.at[0]], ssem).wait()
        pltpu.async_copy(ones, acc.at[dbuf.at[i]], ssem, add=True)
        return 0
    lax.fori_loop(0, RPT, body, 0)
    for _ in range(4):
        pltpu.make_async_copy(ones, acc.at[dbuf.at[0]], ssem).wait()
    plsc.subcore_barrier()

    pltpu.sync_copy(acc.at[pl.ds(r0, NPT), :], out_hbm.at[c, pl.ds(r0, NPT), :])


@functools.partial(
    pl.kernel,
    mesh=_MESH,
    out_type=jax.ShapeDtypeStruct((2, N_PAD, D), jnp.float32),
    scratch_types=[
        pltpu.VMEM((40, 128), jnp.int32),
        pltpu.VMEM((40, 128), jnp.int32),
        pltpu.VMEM((128, D), jnp.float32),
        pltpu.VMEM((128, D), jnp.float32),
        pltpu.VMEM_SHARED((N_PAD, D), jnp.float32),
        pltpu.SemaphoreType.DMA,
        pltpu.SemaphoreType.DMA,
        pltpu.SemaphoreType.DMA,
        pltpu.SemaphoreType.DMA,
    ],
)
def _edge_pass(h_hbm, src_hbm, dst_hbm, zero_hbm, out_hbm,
               sbuf, dbuf, rows0, rows1, acc, gs0, gs1, ss0, ss1):
    c = lax.axis_index("c")
    s = lax.axis_index("s")
    w = c * 16 + s
    C = 40

    r0 = s * NPT
    for q in range(5):
        pltpu.sync_copy(zero_hbm, acc.at[pl.ds(r0 + q * 128, 128), :])
    plsc.subcore_barrier()

    for p in range(RPT // C):
        pltpu.sync_copy(src_hbm.at[w, pl.ds(p * C, C), :], sbuf)
        pltpu.sync_copy(dst_hbm.at[w, pl.ds(p * C, C), :], dbuf)
        pltpu.async_copy(h_hbm.at[sbuf.at[0]], rows0, gs0)

        def pair(j, _):
            i0 = 2 * j
            i1 = 2 * j + 1
            pltpu.make_async_copy(h_hbm.at[sbuf.at[0]], rows0, gs0).wait()
            pltpu.async_copy(rows0, acc.at[dbuf.at[i0]], ss0, add=True)

            @pl.when(j > 0)
            def _():
                pltpu.make_async_copy(rows1, acc.at[dbuf.at[0]], ss1).wait()
            pltpu.async_copy(h_hbm.at[sbuf.at[i1]], rows1, gs1)
            pltpu.make_async_copy(h_hbm.at[sbuf.at[0]], rows1, gs1).wait()
            pltpu.async_copy(rows1, acc.at[dbuf.at[i1]], ss1, add=True)
            pltpu.make_async_copy(rows0, acc.at[dbuf.at[0]], ss0).wait()
            nxt = jnp.minimum(i0 + 2, C - 1)
            pltpu.async_copy(h_hbm.at[sbuf.at[nxt]], rows0, gs0)
            return 0
        lax.fori_loop(0, C // 2, pair, 0)
        pltpu.make_async_copy(rows1, acc.at[dbuf.at[0]], ss1).wait()
        pltpu.make_async_copy(h_hbm.at[sbuf.at[0]], rows0, gs0).wait()
    plsc.subcore_barrier()

    pltpu.sync_copy(acc.at[pl.ds(r0, NPT), :], out_hbm.at[c, pl.ds(r0, NPT), :])


_BLK = 5000
_GRID = N // _BLK


def _dinv_of(deg2_blk):
    deg = deg2_blk[0, :, 0] + deg2_blk[1, :, 0]
    return jnp.where(deg > 0, 1.0 / jnp.sqrt(jnp.maximum(deg, 1.0)), 0.0)


def _tc1_body(deg2_ref, x_ref, w_ref, o_ref, dinv8_ref):
    dinv = _dinv_of(deg2_ref[...])
    h = jnp.dot(x_ref[...], w_ref[...], preferred_element_type=jnp.float32)
    o_ref[...] = h * dinv[:, None]
    dinv8_ref[...] = jnp.broadcast_to(dinv[:, None], (dinv.shape[0], 8))


def _tc2_body(s2_ref, dinv8_ref, b_ref, w_ref, o_ref):
    dinv = dinv8_ref[:, 0]
    t = s2_ref[0] + s2_ref[1]
    t = jnp.maximum(t * dinv[:, None] + b_ref[...], 0.0)
    h = jnp.dot(t, w_ref[...], preferred_element_type=jnp.float32)
    o_ref[...] = h * dinv[:, None]


def _tc3_body(s2_ref, dinv8_ref, b_ref, o_ref):
    dinv = dinv8_ref[:, 0]
    t = s2_ref[0] + s2_ref[1]
    o_ref[...] = jnp.maximum(t * dinv[:, None] + b_ref[...], 0.0)


_deg_spec = pl.BlockSpec((2, _BLK, D), lambda i: (0, i, 0))
_row_spec = pl.BlockSpec((_BLK, D), lambda i: (i, 0))
_s_spec = pl.BlockSpec((2, _BLK, D), lambda i: (0, i, 0))
_w_spec = pl.BlockSpec((D, D), lambda i: (0, 0))
_b_spec = pl.BlockSpec((1, D), lambda i: (0, 0))
_d8_spec = pl.BlockSpec((_BLK, 8), lambda i: (i, 0))
_out_t = jax.ShapeDtypeStruct((N, D), jnp.float32)
_d8_t = jax.ShapeDtypeStruct((N, 8), jnp.float32)

_tc1 = pl.pallas_call(
    _tc1_body, grid=(_GRID,),
    in_specs=[_deg_spec, _row_spec, _w_spec],
    out_specs=[_row_spec, _d8_spec], out_shape=[_out_t, _d8_t])

_tc2 = pl.pallas_call(
    _tc2_body, grid=(_GRID,),
    in_specs=[_s_spec, _d8_spec, _b_spec, _w_spec],
    out_specs=_row_spec, out_shape=_out_t)

_tc3 = pl.pallas_call(
    _tc3_body, grid=(_GRID,),
    in_specs=[_s_spec, _d8_spec, _b_spec],
    out_specs=_row_spec, out_shape=_out_t)


def kernel(x, edge_index, W1, b1, W2, b2):
    src3d = jnp.concatenate(
        [edge_index[0], jnp.asarray(_PAD_SRC)]).reshape(32, RPT, 128)
    dst3d = jnp.concatenate(
        [edge_index[1], jnp.asarray(_PAD_DST)]).reshape(32, RPT, 128)
    ones_c = jnp.asarray(_ONESD)
    zero_c = jnp.asarray(_ZEROD)
    deg2 = _deg_hist(dst3d, ones_c, zero_c)
    h1, dinv8 = _tc1(deg2, x, W1)
    s1 = _edge_pass(h1, src3d, dst3d, zero_c)
    h2 = _tc2(s1, dinv8, b1.reshape(1, D), W2)
    s2 = _edge_pass(h2, src3d, dst3d, zero_c)
    return _tc3(s2, dinv8, b2.reshape(1, D))
